# baseline (speedup 1.0000x reference)
import functools

import jax
import jax.numpy as jnp
from jax import lax
from jax.experimental import pallas as pl
from jax.experimental.pallas import tpu as pltpu
from jax.experimental.pallas import tpu_sc as plsc

_F32 = jnp.float32
_BF16 = jnp.bfloat16

_NC, _NS, _L = 2, 16, 16
_NW = _NC * _NS

_N_ED = 200
_OFF_VA = 200
_NROWS = 800


def _gelu(x):
    return 0.5 * x * (1.0 + jax.lax.erf(x * 0.7071067811865476))


def _tables_kernel(dims, era_ref, dec_ref, vis_ref, aud_ref, Wf1_ref,
                   big_ref):
    (d_era, d_dec, d_year, d_vis, n_era, n_dec, n_vis, n_aud) = dims
    c_dec = d_era
    c_year = c_dec + d_dec
    c_vis = c_year + d_year
    c_aud = c_vis + d_vis

    era_ct = jnp.dot(era_ref[...], Wf1_ref[0:c_dec, :],
                     preferred_element_type=_F32)
    dec_ct = jnp.dot(dec_ref[...], Wf1_ref[c_dec:c_year, :],
                     preferred_element_type=_F32)
    vis_ct = jnp.dot(vis_ref[...], Wf1_ref[c_vis:c_aud, :],
                     preferred_element_type=_F32)
    aud_ct = jnp.dot(aud_ref[...], Wf1_ref[c_aud:, :],
                     preferred_element_type=_F32)

    rE = lax.broadcasted_iota(jnp.int32, (_N_ED, n_era), 0)
    cE = lax.broadcasted_iota(jnp.int32, (_N_ED, n_era), 1)
    rD = lax.broadcasted_iota(jnp.int32, (_N_ED, n_dec), 0)
    cD = lax.broadcasted_iota(jnp.int32, (_N_ED, n_dec), 1)
    big_ref[0:_N_ED, :] = (
        jnp.dot((rE // n_dec == cE).astype(_F32), era_ct,
                preferred_element_type=_F32)
        + jnp.dot((rD % n_dec == cD).astype(_F32), dec_ct,
                  preferred_element_type=_F32))

    n_va = n_vis * n_aud
    rV = lax.broadcasted_iota(jnp.int32, (n_va, n_vis), 0)
    cV = lax.broadcasted_iota(jnp.int32, (n_va, n_vis), 1)
    rA = lax.broadcasted_iota(jnp.int32, (n_va, n_aud), 0)
    cA = lax.broadcasted_iota(jnp.int32, (n_va, n_aud), 1)
    big_ref[_OFF_VA:_OFF_VA + n_va, :] = (
        jnp.dot((rV // n_aud == cV).astype(_F32), vis_ct,
                preferred_element_type=_F32)
        + jnp.dot((rA % n_aud == cA).astype(_F32), aud_ct,
                  preferred_element_type=_F32))


def _sc_gather_kernel(bpw, C, H, tab, e, d, v, a, out,
                      e_v, d_v, v_v, a_v, i1, i2, b0, b1, s0, s1):
    wid = lax.axis_index("s") * _NC + lax.axis_index("c")
    base = wid * bpw
    pltpu.sync_copy(e.at[pl.ds(base, bpw)], e_v)
    pltpu.sync_copy(d.at[pl.ds(base, bpw)], d_v)
    pltpu.sync_copy(v.at[pl.ds(base, bpw)], v_v)
    pltpu.sync_copy(a.at[pl.ds(base, bpw)], a_v)

    def chunk(c2, _):
        r0 = c2 * C
        for s2 in range(C // _L):
            src = pl.ds(r0 + s2 * _L, _L)
            dst = pl.ds(s2 * _L, _L)
            i1[dst] = e_v[src] * 13 + d_v[src]
            i2[dst] = v_v[src] * 20 + a_v[src] + _OFF_VA
        cp0 = pltpu.async_copy(tab.at[i1], b0, s0)
        cp1 = pltpu.async_copy(tab.at[i2], b1, s1)
        cp0.wait()
        cp1.wait()

        def rowbody(r, _):
            for s in range(H // _L):
                sl = pl.ds(s * _L, _L)
                plsc.addupdate(b0.at[r, sl], b1[r, sl])
            return 0

        lax.fori_loop(0, C, rowbody, 0)
        pltpu.sync_copy(b0, out.at[pl.ds(base + r0, C)])
        return 0

    lax.fori_loop(0, bpw // C, chunk, 0)


def _finish_kernel(c_year, d_year, bblk,
                   hp_ref, yr_ref, Wf1_ref, Wy1_ref, by1_ref, Wy2_ref,
                   by2_ref, bf1_ref, Wf2_ref, bf2_ref, out_ref,
                   wy_s, bf_s, wf2_s):
    @pl.when(pl.program_id(0) == 0)
    def _precompute():
        w_year = Wf1_ref[c_year:c_year + d_year, :]
        wy_s[...] = jnp.dot(Wy2_ref[...], w_year, preferred_element_type=_F32)
        bf_s[...] = bf1_ref[...] + jnp.dot(by2_ref[...], w_year,
                                           preferred_element_type=_F32)
        wf2_s[...] = Wf2_ref[...].astype(_BF16)

    yn = (yr_ref[...].astype(_F32) - 1920.0) / 110.0
    y1 = _gelu(yn * Wy1_ref[...] + by1_ref[...])
    acc = hp_ref[...] + jnp.dot(y1, wy_s[...], preferred_element_type=_F32)
    acc = acc + bf_s[...]
    h = _gelu(acc)
    out_ref[...] = jnp.dot(h.astype(_BF16), wf2_s[...],
                           preferred_element_type=_F32) + bf2_ref[...]


def kernel(era_ids, decade_ids, years, visual_styles, audio_styles,
           era_table, decade_table, visual_table, audio_table,
           W_y1, b_y1, W_y2, b_y2, W_f1, b_f1, W_f2, b_f2):
    B = era_ids.shape[0]
    n_era, d_era = era_table.shape
    n_dec, d_dec = decade_table.shape
    n_vis, d_vis = visual_table.shape
    n_aud, d_aud = audio_table.shape
    d_year = W_y1.shape[1]
    d_in = d_era + d_dec + d_year + d_vis + d_aud
    c_year = d_era + d_dec
    H = W_f2.shape[1]
    dims = (d_era, d_dec, d_year, d_vis, n_era, n_dec, n_vis, n_aud)

    full = lambda shape: pl.BlockSpec(shape, lambda *_: tuple(0 for _ in shape))
    big = pl.pallas_call(
        functools.partial(_tables_kernel, dims),
        in_specs=[full(era_table.shape), full(decade_table.shape),
                  full(visual_table.shape), full(audio_table.shape),
                  full((d_in, H))],
        out_specs=full((_NROWS, H)),
        out_shape=jax.ShapeDtypeStruct((_NROWS, H), _F32),
    )(era_table, decade_table, visual_table, audio_table, W_f1)

    bpw = B // _NW
    C = 64
    mesh = plsc.VectorSubcoreMesh(core_axis_name="c", subcore_axis_name="s")
    h_pre = pl.kernel(
        functools.partial(_sc_gather_kernel, bpw, C, H),
        out_type=jax.ShapeDtypeStruct((B, H), _F32),
        mesh=mesh,
        scratch_types=[
            pltpu.VMEM((bpw,), jnp.int32), pltpu.VMEM((bpw,), jnp.int32),
            pltpu.VMEM((bpw,), jnp.int32), pltpu.VMEM((bpw,), jnp.int32),
            pltpu.VMEM((C,), jnp.int32), pltpu.VMEM((C,), jnp.int32),
            pltpu.VMEM((C, H), _F32), pltpu.VMEM((C, H), _F32),
            pltpu.SemaphoreType.DMA, pltpu.SemaphoreType.DMA,
        ],
    )(big, era_ids.astype(jnp.int32), decade_ids.astype(jnp.int32),
      visual_styles.astype(jnp.int32), audio_styles.astype(jnp.int32))

    bblk = 2048
    out = pl.pallas_call(
        functools.partial(_finish_kernel, c_year, d_year, bblk),
        grid=(B // bblk,),
        in_specs=[pl.BlockSpec((bblk, H), lambda i: (i, 0)),
                  pl.BlockSpec((bblk, 1), lambda i: (i, 0)),
                  full((d_in, H)), full((1, d_year)), full((1, d_year)),
                  full((d_year, d_year)), full((1, d_year)), full((1, H)),
                  full((H, H)), full((1, H))],
        out_specs=pl.BlockSpec((bblk, H), lambda i: (i, 0)),
        out_shape=jax.ShapeDtypeStruct((B, H), _F32),
        scratch_shapes=[
            pltpu.VMEM((d_year, H), _F32),
            pltpu.VMEM((1, H), _F32),
            pltpu.VMEM((H, H), _BF16),
        ],
    )(h_pre, years.astype(jnp.int32).reshape(B, 1), W_f1, W_y1,
      b_y1.reshape(1, d_year), W_y2, b_y2.reshape(1, d_year),
      b_f1.reshape(1, H), W_f2, b_f2.reshape(1, H))
    return out


# speedup vs baseline: 2.8004x; 2.8004x over previous
import functools

import jax
import jax.numpy as jnp
from jax.experimental import pallas as pl
from jax.experimental.pallas import tpu as pltpu

_F32 = jnp.float32
_BF16 = jnp.bfloat16

_R_ERA, _R_DEC, _R_VIS, _R_AUD, _NROWS = 0, 16, 32, 64, 128


def _gelu(x):
    return 0.5 * x * (1.0 + jax.lax.erf(x * 0.7071067811865476))


def _era_kernel(dims, bblk,
                ids_ref,
                era_ref, dec_ref, vis_ref, aud_ref, Wf1_ref,
                Wy1_ref, by1_ref, Wy2_ref, by2_ref, bf1_ref,
                Wf2_ref, bf2_ref, out_ref, cat_s, wy_s, bf_s, wf2_s):
    (d_era, d_dec, d_year, d_vis, d_aud, n_era, n_dec, n_vis, n_aud) = dims
    c_dec = d_era
    c_year = c_dec + d_dec
    c_vis = c_year + d_year
    c_aud = c_vis + d_vis

    @pl.when(pl.program_id(0) == 0)
    def _precompute():
        cat_s[...] = jnp.zeros((_NROWS, cat_s.shape[1]), _F32)
        cat_s[_R_ERA:_R_ERA + n_era, :] = jnp.dot(
            era_ref[...], Wf1_ref[0:c_dec, :], preferred_element_type=_F32)
        cat_s[_R_DEC:_R_DEC + n_dec, :] = jnp.dot(
            dec_ref[...], Wf1_ref[c_dec:c_year, :],
            preferred_element_type=_F32)
        cat_s[_R_VIS:_R_VIS + n_vis, :] = jnp.dot(
            vis_ref[...], Wf1_ref[c_vis:c_aud, :],
            preferred_element_type=_F32)
        cat_s[_R_AUD:_R_AUD + n_aud, :] = jnp.dot(
            aud_ref[...], Wf1_ref[c_aud:, :], preferred_element_type=_F32)
        w_year = Wf1_ref[c_year:c_vis, :]
        wy_s[...] = jnp.dot(Wy2_ref[...], w_year, preferred_element_type=_F32)
        bf_s[...] = bf1_ref[...] + jnp.dot(by2_ref[...], w_year,
                                           preferred_element_type=_F32)
        wf2_s[...] = Wf2_ref[...].astype(_BF16)

    ids = ids_ref[...]
    iot = jax.lax.broadcasted_iota(jnp.int32, (bblk, _NROWS), 1)
    oh = ((iot == ids[:, 0:1])
          | (iot == ids[:, 1:2] + _R_DEC)
          | (iot == ids[:, 2:3] + _R_VIS)
          | (iot == ids[:, 3:4] + _R_AUD)).astype(_F32)

    yn = (ids[:, 4:5].astype(_F32) - 1920.0) / 110.0
    y1 = _gelu(yn * Wy1_ref[...] + by1_ref[...])

    acc = jnp.dot(oh, cat_s[...], preferred_element_type=_F32)
    acc = acc + jnp.dot(y1, wy_s[...], preferred_element_type=_F32)
    acc = acc + bf_s[...]
    h = _gelu(acc)
    out_ref[...] = jnp.dot(h.astype(_BF16), wf2_s[...],
                           preferred_element_type=_F32) + bf2_ref[...]


def kernel(era_ids, decade_ids, years, visual_styles, audio_styles,
           era_table, decade_table, visual_table, audio_table,
           W_y1, b_y1, W_y2, b_y2, W_f1, b_f1, W_f2, b_f2):
    B = era_ids.shape[0]
    n_era, d_era = era_table.shape
    n_dec, d_dec = decade_table.shape
    n_vis, d_vis = visual_table.shape
    n_aud, d_aud = audio_table.shape
    d_year = W_y1.shape[1]
    d_in = d_era + d_dec + d_year + d_vis + d_aud
    H = W_f2.shape[1]
    dims = (d_era, d_dec, d_year, d_vis, d_aud, n_era, n_dec, n_vis, n_aud)

    i32 = lambda x: x.astype(jnp.int32)
    ids = jnp.stack(
        [i32(era_ids), i32(decade_ids), i32(visual_styles),
         i32(audio_styles), i32(years), i32(years), i32(years), i32(years)],
        axis=1)
    bblk = 1024
    grid = (B // bblk,)

    full = lambda shape: pl.BlockSpec(shape, lambda i: (0, 0))
    out = pl.pallas_call(
        functools.partial(_era_kernel, dims, bblk),
        grid=grid,
        in_specs=[
            pl.BlockSpec((bblk, 8), lambda i: (i, 0)),
            full(era_table.shape), full(decade_table.shape),
            full(visual_table.shape), full(audio_table.shape),
            full((d_in, H)),
            full((1, d_year)),
            full((1, d_year)),
            full((d_year, d_year)),
            full((1, d_year)),
            full((1, H)),
            full((H, H)),
            full((1, H)),
        ],
        out_specs=pl.BlockSpec((bblk, H), lambda i: (i, 0)),
        out_shape=jax.ShapeDtypeStruct((B, H), _F32),
        scratch_shapes=[
            pltpu.VMEM((_NROWS, H), _F32),
            pltpu.VMEM((d_year, H), _F32),
            pltpu.VMEM((1, H), _F32),
            pltpu.VMEM((H, H), _BF16),
        ],
    )(ids, era_table, decade_table, visual_table, audio_table,
      W_f1, W_y1, b_y1.reshape(1, d_year), W_y2, b_y2.reshape(1, d_year),
      b_f1.reshape(1, H), W_f2, b_f2.reshape(1, H))
    return out


# speedup vs baseline: 2.9942x; 1.0692x over previous
import functools

import jax
import jax.numpy as jnp
from jax.experimental import pallas as pl
from jax.experimental.pallas import tpu as pltpu

_F32 = jnp.float32
_BF16 = jnp.bfloat16

_R_ERA, _R_DEC, _R_VIS, _R_AUD, _NROWS = 0, 16, 32, 64, 128


def _gelu(x):
    return 0.5 * x * (1.0 + jax.lax.erf(x * 0.7071067811865476))


def _era_kernel(dims, bblk,
                ids_ref,
                era_ref, dec_ref, vis_ref, aud_ref, Wf1_ref,
                Wy1_ref, by1_ref, Wy2_ref, by2_ref, bf1_ref,
                Wf2_ref, bf2_ref, out_ref, cat_s, wy_s, bf_s, wf2_s):
    (d_era, d_dec, d_year, d_vis, d_aud, n_era, n_dec, n_vis, n_aud) = dims
    c_dec = d_era
    c_year = c_dec + d_dec
    c_vis = c_year + d_year
    c_aud = c_vis + d_vis

    @pl.when(pl.program_id(0) == 0)
    def _precompute():
        cat_s[...] = jnp.zeros((_NROWS, cat_s.shape[1]), _F32)
        cat_s[_R_ERA:_R_ERA + n_era, :] = jnp.dot(
            era_ref[...], Wf1_ref[0:c_dec, :], preferred_element_type=_F32)
        cat_s[_R_DEC:_R_DEC + n_dec, :] = jnp.dot(
            dec_ref[...], Wf1_ref[c_dec:c_year, :],
            preferred_element_type=_F32)
        cat_s[_R_VIS:_R_VIS + n_vis, :] = jnp.dot(
            vis_ref[...], Wf1_ref[c_vis:c_aud, :],
            preferred_element_type=_F32)
        cat_s[_R_AUD:_R_AUD + n_aud, :] = jnp.dot(
            aud_ref[...], Wf1_ref[c_aud:, :], preferred_element_type=_F32)
        w_year = Wf1_ref[c_year:c_vis, :]
        wy_s[...] = jnp.dot(Wy2_ref[...], w_year, preferred_element_type=_F32)
        bf_s[...] = bf1_ref[...] + jnp.dot(by2_ref[...], w_year,
                                           preferred_element_type=_F32)
        wf2_s[...] = Wf2_ref[...].astype(_BF16)

    ids = ids_ref[...]
    iot = jax.lax.broadcasted_iota(jnp.int32, (bblk, _NROWS), 1)
    oh = ((iot == ids[:, 0:1])
          | (iot == ids[:, 1:2] + _R_DEC)
          | (iot == ids[:, 2:3] + _R_VIS)
          | (iot == ids[:, 3:4] + _R_AUD)).astype(_F32)

    yn = (ids[:, 4:5].astype(_F32) - 1920.0) / 110.0
    y1 = _gelu(yn * Wy1_ref[...] + by1_ref[...])

    acc = jnp.dot(oh, cat_s[...], preferred_element_type=_F32)
    acc = acc + jnp.dot(y1, wy_s[...], preferred_element_type=_F32)
    acc = acc + bf_s[...]
    h = _gelu(acc)
    out_ref[...] = jnp.dot(h.astype(_BF16), wf2_s[...],
                           preferred_element_type=_F32) + bf2_ref[...]


def kernel(era_ids, decade_ids, years, visual_styles, audio_styles,
           era_table, decade_table, visual_table, audio_table,
           W_y1, b_y1, W_y2, b_y2, W_f1, b_f1, W_f2, b_f2):
    B = era_ids.shape[0]
    n_era, d_era = era_table.shape
    n_dec, d_dec = decade_table.shape
    n_vis, d_vis = visual_table.shape
    n_aud, d_aud = audio_table.shape
    d_year = W_y1.shape[1]
    d_in = d_era + d_dec + d_year + d_vis + d_aud
    H = W_f2.shape[1]
    dims = (d_era, d_dec, d_year, d_vis, d_aud, n_era, n_dec, n_vis, n_aud)

    i32 = lambda x: x.astype(jnp.int32)
    ids = jnp.stack(
        [i32(era_ids), i32(decade_ids), i32(visual_styles),
         i32(audio_styles), i32(years), i32(years), i32(years), i32(years)],
        axis=1)
    bblk = 2048
    grid = (B // bblk,)

    full = lambda shape: pl.BlockSpec(shape, lambda i: (0, 0))
    out = pl.pallas_call(
        functools.partial(_era_kernel, dims, bblk),
        grid=grid,
        in_specs=[
            pl.BlockSpec((bblk, 8), lambda i: (i, 0)),
            full(era_table.shape), full(decade_table.shape),
            full(visual_table.shape), full(audio_table.shape),
            full((d_in, H)),
            full((1, d_year)),
            full((1, d_year)),
            full((d_year, d_year)),
            full((1, d_year)),
            full((1, H)),
            full((H, H)),
            full((1, H)),
        ],
        out_specs=pl.BlockSpec((bblk, H), lambda i: (i, 0)),
        out_shape=jax.ShapeDtypeStruct((B, H), _F32),
        scratch_shapes=[
            pltpu.VMEM((_NROWS, H), _F32),
            pltpu.VMEM((d_year, H), _F32),
            pltpu.VMEM((1, H), _F32),
            pltpu.VMEM((H, H), _BF16),
        ],
    )(ids, era_table, decade_table, visual_table, audio_table,
      W_f1, W_y1, b_y1.reshape(1, d_year), W_y2, b_y2.reshape(1, d_year),
      b_f1.reshape(1, H), W_f2, b_f2.reshape(1, H))
    return out
